# baseline (speedup 1.0000x reference)
import jax
import jax.numpy as jnp
from jax import lax
from jax.experimental import pallas as pl
from jax.experimental.pallas import tpu as pltpu
from jax.experimental.pallas import tpu_sc as plsc

VOCAB = 1000000
DIM = 64
BATCH = 16384
CTX = 20

NC = 2
NS = 16
NW = NC * NS
WPW = BATCH // NW
WCH = WPW // 128
CPW = BATCH * CTX // NW
CCH = CPW // 128


def _sgns_gather(widx_hbm, cidx_hbm, w_tbl, c_tbl, out_w, out_c,
                 widx_v, cidx_v, rows0, rows1, sem0, sem1):
    wid = lax.axis_index("s") * NC + lax.axis_index("c")

    pltpu.sync_copy(widx_hbm.at[pl.ds(wid * WCH, WCH)], widx_v)
    pltpu.sync_copy(cidx_hbm.at[pl.ds(wid * CCH, CCH)], cidx_v)

    rows = (rows0, rows1)
    sems = (sem0, sem1)

    pltpu.async_copy(w_tbl.at[widx_v.at[0]], rows0, sem0)
    for j in range(WCH):
        b = j % 2
        if j + 1 < WCH:
            pltpu.async_copy(w_tbl.at[widx_v.at[j + 1]],
                             rows[1 - b], sems[1 - b])
        pltpu.make_async_copy(w_tbl.at[widx_v.at[j]], rows[b],
                              sems[b]).wait()
        pltpu.sync_copy(rows[b],
                        out_w.at[pl.ds(wid * WPW + j * 128, 128)])

    pltpu.async_copy(c_tbl.at[cidx_v.at[0]], rows0, sem0)

    def c_body(g, carry):
        j0 = 2 * g
        pltpu.async_copy(c_tbl.at[cidx_v.at[j0 + 1]], rows1, sem1)
        pltpu.make_async_copy(c_tbl.at[cidx_v.at[j0]], rows0, sem0).wait()
        pltpu.sync_copy(rows0,
                        out_c.at[pl.ds(wid * CPW + j0 * 128, 128)])

        @pl.when(j0 + 2 < CCH)
        def _():
            pltpu.async_copy(c_tbl.at[cidx_v.at[j0 + 2]], rows0, sem0)

        pltpu.make_async_copy(c_tbl.at[cidx_v.at[j0 + 1]], rows1,
                              sem1).wait()
        pltpu.sync_copy(rows1,
                        out_c.at[pl.ds(wid * CPW + (j0 + 1) * 128, 128)])
        return carry

    lax.fori_loop(0, CCH // 2, c_body, 0)


@jax.jit
def _run(widx, cidx, w_tbl, c_tbl):
    mesh = plsc.VectorSubcoreMesh(core_axis_name="c", subcore_axis_name="s")
    return pl.kernel(
        _sgns_gather,
        mesh=mesh,
        out_type=[
            jax.ShapeDtypeStruct((BATCH, DIM), jnp.float32),
            jax.ShapeDtypeStruct((BATCH * CTX, DIM), jnp.float32),
        ],
        scratch_types=[
            pltpu.VMEM((WCH, 128), jnp.int32),
            pltpu.VMEM((CCH, 128), jnp.int32),
            pltpu.VMEM((128, DIM), jnp.float32),
            pltpu.VMEM((128, DIM), jnp.float32),
            pltpu.SemaphoreType.DMA,
            pltpu.SemaphoreType.DMA,
        ],
        compiler_params=pltpu.CompilerParams(use_tc_tiling_on_sc=False),
    )(widx, cidx, w_tbl, c_tbl)


def kernel(words, contexts, w_embedding, c_embedding):
    widx = words.reshape(BATCH // 128, 128)
    cidx = contexts.reshape(BATCH * CTX // 128, 128)
    out_w, out_c = _run(widx, cidx, w_embedding, c_embedding)
    return out_w, out_c.reshape(BATCH, CTX, DIM)


# speedup vs baseline: 1.0333x; 1.0333x over previous
import jax
import jax.numpy as jnp
from jax import lax
from jax.experimental import pallas as pl
from jax.experimental.pallas import tpu as pltpu
from jax.experimental.pallas import tpu_sc as plsc

VOCAB = 1000000
DIM = 64
BATCH = 16384
CTX = 20

NC = 2
NS = 16
NW = NC * NS
WPW = BATCH // NW
WCH = WPW // 128
CPW = BATCH * CTX // NW
CCH = CPW // 128


def _make_gather(n_chunks, rows_per_worker):
    def body(idx_hbm, tbl, out, idx_v, rows0, rows1, sem0, sem1):
        wid = lax.axis_index("s") * NC + lax.axis_index("c")

        pltpu.sync_copy(idx_hbm.at[pl.ds(wid * n_chunks, n_chunks)], idx_v)

        pltpu.async_copy(tbl.at[idx_v.at[0]], rows0, sem0)

        def c_body(g, carry):
            j0 = 2 * g
            pltpu.async_copy(tbl.at[idx_v.at[j0 + 1]], rows1, sem1)
            pltpu.make_async_copy(tbl.at[idx_v.at[j0]], rows0, sem0).wait()
            pltpu.sync_copy(
                rows0, out.at[pl.ds(wid * rows_per_worker + j0 * 128, 128)])

            @pl.when(j0 + 2 < n_chunks)
            def _():
                pltpu.async_copy(tbl.at[idx_v.at[j0 + 2]], rows0, sem0)

            pltpu.make_async_copy(tbl.at[idx_v.at[j0 + 1]], rows1,
                                  sem1).wait()
            pltpu.sync_copy(
                rows1,
                out.at[pl.ds(wid * rows_per_worker + (j0 + 1) * 128, 128)])
            return carry

        lax.fori_loop(0, n_chunks // 2, c_body, 0)

    return body


def _gather_call(idx, tbl, n_rows, n_chunks, rows_per_worker):
    mesh = plsc.VectorSubcoreMesh(core_axis_name="c", subcore_axis_name="s")
    return pl.kernel(
        _make_gather(n_chunks, rows_per_worker),
        mesh=mesh,
        out_type=jax.ShapeDtypeStruct((n_rows, DIM), jnp.float32),
        scratch_types=[
            pltpu.VMEM((n_chunks, 128), jnp.int32),
            pltpu.VMEM((128, DIM), jnp.float32),
            pltpu.VMEM((128, DIM), jnp.float32),
            pltpu.SemaphoreType.DMA,
            pltpu.SemaphoreType.DMA,
        ],
        compiler_params=pltpu.CompilerParams(use_tc_tiling_on_sc=False),
    )(idx, tbl)


@jax.jit
def _run(widx, cidx, w_tbl, c_tbl):
    out_w = _gather_call(widx, w_tbl, BATCH, WCH, WPW)
    out_c = _gather_call(cidx, c_tbl, BATCH * CTX, CCH, CPW)
    return out_w, out_c


def kernel(words, contexts, w_embedding, c_embedding):
    widx = words.reshape(BATCH // 128, 128)
    cidx = contexts.reshape(BATCH * CTX // 128, 128)
    out_w, out_c = _run(widx, cidx, w_embedding, c_embedding)
    return out_w, out_c.reshape(BATCH, CTX, DIM)
